# baseline (speedup 1.0000x reference)
import functools

import jax
import jax.numpy as jnp
from jax import lax
from jax.experimental import pallas as pl
from jax.experimental.pallas import tpu as pltpu
from jax.experimental.pallas import tpu_sc as plsc

_N = 10000
_E = 320000
_D = 128
_L = 4
_ALPHA = 0.1
_THETA = 0.5

_NC = 2
_NS = 16
_NW = _NC * _NS
_B = 128

_NP = 10240
_NB = 80
_PT = _NB * _B
_EP = _PT * _NW
_RPT = _NP // _NS
_TB = _EP // _B
_NB0 = _TB // _NS
_SB0 = 80
_SB1 = _NB0 - _SB0
_CB0 = _NS * _SB0

_BN = 1024
_GRID = _NP // _BN


def _mesh():
    return plsc.VectorSubcoreMesh(
        core_axis_name="c", subcore_axis_name="s",
        num_cores=_NC, num_subcores=_NS)


@functools.partial(
    pl.kernel,
    out_type=jax.ShapeDtypeStruct((_NP,), jnp.float32),
    mesh=_mesh(),
    scratch_types=[
        pltpu.VMEM_SHARED((_NP,), jnp.float32),
        pltpu.VMEM((_NB0, _B), jnp.int32),
        pltpu.VMEM((_B,), jnp.float32),
        pltpu.SemaphoreType.DMA,
    ],
)
def _deg_kernel(colp, zcol, onesv, out, acc, col_all, oneb, sem):
    c = lax.axis_index("c")
    s = lax.axis_index("s")

    @pl.when(c == 0)
    def _():
        pltpu.sync_copy(zcol, acc.at[pl.ds(s * _RPT, _RPT)])
        pltpu.sync_copy(onesv, oneb)
        pltpu.sync_copy(colp.at[s], col_all)
        plsc.subcore_barrier()
        descs = [pltpu.async_copy(oneb, acc.at[col_all.at[b]], sem, add=True)
                 for b in range(_NB0)]
        for d in descs:
            d.wait()
        plsc.subcore_barrier()
        pltpu.sync_copy(acc.at[pl.ds(s * _RPT, _RPT)],
                        out.at[pl.ds(s * _RPT, _RPT)])


@functools.partial(
    pl.kernel,
    out_type=jax.ShapeDtypeStruct((_NC, _NP, _D), jnp.float32),
    mesh=_mesh(),
    scratch_types=[
        pltpu.VMEM_SHARED((_NP, _D), jnp.float32),
        pltpu.VMEM((_B, _D), jnp.float32),
        pltpu.VMEM((_B,), jnp.int32),
        pltpu.VMEM((_B,), jnp.int32),
        pltpu.SemaphoreType.DMA,
    ],
)
def _spmm_kernel(hs, rowp, colp, zrows, out, acc, rows, colb, rowb, gsem):
    c = lax.axis_index("c")
    s = lax.axis_index("s")
    pltpu.sync_copy(zrows, acc.at[pl.ds(s * _RPT, _RPT)])
    plsc.subcore_barrier()

    def make_body(base):
        def body(b, carry):
            off = base + b * _B
            pltpu.sync_copy(colp.at[pl.ds(off, _B)], colb)
            pltpu.async_copy(hs.at[colb], rows, gsem).wait()
            pltpu.sync_copy(rowp.at[pl.ds(off, _B)], rowb)
            pltpu.sync_copy(rows, acc.at[rowb], add=True)
            return carry
        return body

    @pl.when(c == 0)
    def _():
        lax.fori_loop(0, _SB0, make_body(s * (_SB0 * _B)), 0)

    @pl.when(c == 1)
    def _():
        lax.fori_loop(0, _SB1, make_body(_CB0 * _B + s * (_SB1 * _B)), 0)

    plsc.subcore_barrier()
    pltpu.sync_copy(acc.at[pl.ds(s * _RPT, _RPT)],
                    out.at[c, pl.ds(s * _RPT, _RPT)])


def _init_body(x_ref, w0_ref, b0_ref, d0_ref, x0_ref, hs_ref, dv_ref):
    h = jnp.dot(x_ref[...], w0_ref[...],
                preferred_element_type=jnp.float32,
                precision=lax.Precision.HIGHEST)
    h = jnp.maximum(h + b0_ref[...], 0.0)
    dv = lax.rsqrt(d0_ref[...] + 1.0)
    x0_ref[...] = h
    hs_ref[...] = h * dv
    dv_ref[...] = dv


def _layer_body(beta, last, s2_ref, hs_ref, x0_ref, dv_ref, w_ref,
                w1_ref, b1_ref, out_ref):
    agg = (s2_ref[0] + s2_ref[1] + hs_ref[...]) * dv_ref[...]
    z = (1.0 - _ALPHA) * agg + _ALPHA * x0_ref[...]
    zw = jnp.dot(z, w_ref[...], preferred_element_type=jnp.float32,
                 precision=lax.Precision.HIGHEST)
    h = jnp.maximum((1.0 - beta) * z + beta * zw, 0.0)
    if last:
        y = jnp.dot(h, w1_ref[...], preferred_element_type=jnp.float32,
                    precision=lax.Precision.HIGHEST)
        out_ref[...] = y + b1_ref[...]
    else:
        out_ref[...] = h * dv_ref[...]


_ROWS = pl.BlockSpec((_BN, _D), lambda i: (i, 0))
_COL1 = pl.BlockSpec((_BN, 1), lambda i: (i, 0))
_FULL = pl.BlockSpec((_D, _D), lambda i: (0, 0))
_BIAS = pl.BlockSpec((1, _D), lambda i: (0, 0))
_S2 = pl.BlockSpec((_NC, _BN, _D), lambda i: (0, i, 0))


def _init_call(x_p, W0, b0, d0):
    return pl.pallas_call(
        _init_body,
        grid=(_GRID,),
        in_specs=[_ROWS, _FULL, _BIAS, _COL1],
        out_specs=[_ROWS, _ROWS, _COL1],
        out_shape=[
            jax.ShapeDtypeStruct((_NP, _D), jnp.float32),
            jax.ShapeDtypeStruct((_NP, _D), jnp.float32),
            jax.ShapeDtypeStruct((_NP, 1), jnp.float32),
        ],
    )(x_p, W0, b0, d0)


def _layer_call(beta, last, s2, hs, x0, dv, W, W1, b1):
    return pl.pallas_call(
        functools.partial(_layer_body, beta, last),
        grid=(_GRID,),
        in_specs=[_S2, _ROWS, _ROWS, _COL1, _FULL, _FULL, _BIAS],
        out_specs=_ROWS,
        out_shape=jax.ShapeDtypeStruct((_NP, _D), jnp.float32),
    )(s2, hs, x0, dv, W, W1, b1)


def kernel(x, edge_index, W0, b0, convW, W1, b1):
    import numpy as np

    row = edge_index[0]
    col = edge_index[1]
    pad = jnp.full((_EP - _E,), _N, dtype=jnp.int32)
    rowp = jnp.concatenate([row, pad])
    colf = jnp.concatenate([col, pad])
    colp = colf.reshape(_NS, _NB0, _B)
    x_p = jnp.pad(x, ((0, _NP - _N), (0, 0)))

    zcol = jnp.zeros((_RPT,), jnp.float32)
    onesv = jnp.ones((_B,), jnp.float32)
    zrows = jnp.zeros((_RPT, _D), jnp.float32)

    deg = _deg_kernel(colp, zcol, onesv)
    d0 = deg[:, None]

    x0, hs, dv = _init_call(x_p, W0, b0[None, :], d0)

    b1r = b1[None, :]
    for layer in range(_L):
        s2 = _spmm_kernel(hs, rowp, colf, zrows)
        beta = float(np.log(_THETA / (layer + 1) + 1.0))
        hs = _layer_call(beta, layer == _L - 1, s2, hs, x0, dv,
                         convW[layer], W1, b1r)
    return hs[:_N]


# speedup vs baseline: 1.1068x; 1.1068x over previous
import functools

import jax
import jax.numpy as jnp
from jax import lax
from jax.experimental import pallas as pl
from jax.experimental.pallas import tpu as pltpu
from jax.experimental.pallas import tpu_sc as plsc

_N = 10000
_E = 320000
_D = 128
_L = 4
_ALPHA = 0.1
_THETA = 0.5

_NC = 2
_NS = 16
_NW = _NC * _NS
_B = 128

_NP = 10240
_NB = 80
_PT = _NB * _B
_EP = _PT * _NW
_RPT = _NP // _NS
_TB = _EP // _B
_NB0 = _TB // _NS
_SB0 = 80
_SB1 = _NB0 - _SB0
_CB0 = _NS * _SB0

_BN = 1024
_GRID = _NP // _BN


def _mesh():
    return plsc.VectorSubcoreMesh(
        core_axis_name="c", subcore_axis_name="s",
        num_cores=_NC, num_subcores=_NS)


@functools.partial(
    pl.kernel,
    out_type=jax.ShapeDtypeStruct((_NP,), jnp.float32),
    mesh=_mesh(),
    scratch_types=[
        pltpu.VMEM_SHARED((_NP,), jnp.float32),
        pltpu.VMEM((_NB0, _B), jnp.int32),
        pltpu.VMEM((_B,), jnp.float32),
        pltpu.SemaphoreType.DMA,
    ],
)
def _deg_kernel(colp, zcol, onesv, out, acc, col_all, oneb, sem):
    c = lax.axis_index("c")
    s = lax.axis_index("s")

    @pl.when(c == 0)
    def _():
        pltpu.sync_copy(zcol, acc.at[pl.ds(s * _RPT, _RPT)])
        pltpu.sync_copy(onesv, oneb)
        pltpu.sync_copy(colp.at[s], col_all)
        plsc.subcore_barrier()
        descs = [pltpu.async_copy(oneb, acc.at[col_all.at[b]], sem, add=True)
                 for b in range(_NB0)]
        for d in descs:
            d.wait()
        plsc.subcore_barrier()
        pltpu.sync_copy(acc.at[pl.ds(s * _RPT, _RPT)],
                        out.at[pl.ds(s * _RPT, _RPT)])


@functools.partial(
    pl.kernel,
    out_type=jax.ShapeDtypeStruct((_NP, _D), jnp.float32),
    mesh=_mesh(),
    scratch_types=[
        pltpu.VMEM_SHARED((_NP, _D), jnp.float32),
        pltpu.VMEM((_B, _D), jnp.float32),
        pltpu.VMEM((_B, _D), jnp.float32),
        pltpu.VMEM((2, _B), jnp.int32),
        pltpu.VMEM((2, _B), jnp.int32),
        pltpu.SemaphoreType.DMA,
        pltpu.SemaphoreType.DMA,
    ],
)
def _spmm_kernel(hs, idxp, zrows, out,
                 acc, rows0, rows1, ib0, ib1, gs0, gs1):
    c = lax.axis_index("c")
    s = lax.axis_index("s")
    base = s * _NB0
    npair = jnp.where(c == 0, _NB0 // 2, 0)

    @pl.when(c == 0)
    def _():
        pltpu.sync_copy(zrows, acc.at[pl.ds(s * _RPT, _RPT)])
        pltpu.sync_copy(idxp.at[base], ib0)
        pltpu.sync_copy(idxp.at[base + 1], ib1)
        pltpu.async_copy(hs.at[ib0.at[0]], rows0, gs0)

    plsc.subcore_barrier()

    def pair(p, carry):
        b0 = base + 2 * p
        g1 = pltpu.async_copy(hs.at[ib1.at[0]], rows1, gs1)
        pltpu.make_async_copy(hs.at[ib0.at[0]], rows0, gs0).wait()
        pltpu.sync_copy(rows0, acc.at[ib0.at[1]], add=True)

        @pl.when(p < npair - 1)
        def _():
            pltpu.sync_copy(idxp.at[b0 + 2], ib0)
            pltpu.async_copy(hs.at[ib0.at[0]], rows0, gs0)

        g1.wait()
        pltpu.sync_copy(rows1, acc.at[ib1.at[1]], add=True)

        @pl.when(p < npair - 1)
        def _():
            pltpu.sync_copy(idxp.at[b0 + 3], ib1)

        return carry

    lax.fori_loop(0, npair, pair, 0)
    plsc.subcore_barrier()

    @pl.when(c == 0)
    def _():
        pltpu.sync_copy(acc.at[pl.ds(s * _RPT, _RPT)],
                        out.at[pl.ds(s * _RPT, _RPT)])


def _init_body(x_ref, w0_ref, b0_ref, d0_ref, x0_ref, hs_ref, dv_ref):
    h = jnp.dot(x_ref[...], w0_ref[...],
                preferred_element_type=jnp.float32,
                precision=lax.Precision.HIGHEST)
    h = jnp.maximum(h + b0_ref[...], 0.0)
    dv = lax.rsqrt(d0_ref[...] + 1.0)
    x0_ref[...] = h
    hs_ref[...] = h * dv
    dv_ref[...] = dv


def _layer_body(beta, last, s_ref, hs_ref, x0_ref, dv_ref, w_ref,
                w1_ref, b1_ref, out_ref):
    agg = (s_ref[...] + hs_ref[...]) * dv_ref[...]
    z = (1.0 - _ALPHA) * agg + _ALPHA * x0_ref[...]
    zw = jnp.dot(z, w_ref[...], preferred_element_type=jnp.float32,
                 precision=lax.Precision.HIGHEST)
    h = jnp.maximum((1.0 - beta) * z + beta * zw, 0.0)
    if last:
        y = jnp.dot(h, w1_ref[...], preferred_element_type=jnp.float32,
                    precision=lax.Precision.HIGHEST)
        out_ref[...] = y + b1_ref[...]
    else:
        out_ref[...] = h * dv_ref[...]


_ROWS = pl.BlockSpec((_BN, _D), lambda i: (i, 0))
_COL1 = pl.BlockSpec((_BN, 1), lambda i: (i, 0))
_FULL = pl.BlockSpec((_D, _D), lambda i: (0, 0))
_BIAS = pl.BlockSpec((1, _D), lambda i: (0, 0))
_S2 = pl.BlockSpec((_NC, _BN, _D), lambda i: (0, i, 0))


def _init_call(x_p, W0, b0, d0):
    return pl.pallas_call(
        _init_body,
        grid=(_GRID,),
        in_specs=[_ROWS, _FULL, _BIAS, _COL1],
        out_specs=[_ROWS, _ROWS, _COL1],
        out_shape=[
            jax.ShapeDtypeStruct((_NP, _D), jnp.float32),
            jax.ShapeDtypeStruct((_NP, _D), jnp.float32),
            jax.ShapeDtypeStruct((_NP, 1), jnp.float32),
        ],
    )(x_p, W0, b0, d0)


def _layer_call(beta, last, s, hs, x0, dv, W, W1, b1):
    return pl.pallas_call(
        functools.partial(_layer_body, beta, last),
        grid=(_GRID,),
        in_specs=[_ROWS, _ROWS, _ROWS, _COL1, _FULL, _FULL, _BIAS],
        out_specs=_ROWS,
        out_shape=jax.ShapeDtypeStruct((_NP, _D), jnp.float32),
    )(s, hs, x0, dv, W, W1, b1)


def kernel(x, edge_index, W0, b0, convW, W1, b1):
    import numpy as np

    row = edge_index[0]
    col = edge_index[1]
    pad = jnp.full((_EP - _E,), _N, dtype=jnp.int32)
    rowf = jnp.concatenate([row, pad]).reshape(_TB, _B)
    colf = jnp.concatenate([col, pad]).reshape(_TB, _B)
    idxp = jnp.stack([colf, rowf], axis=1)
    colp = colf.reshape(_NS, _NB0, _B)
    x_p = jnp.pad(x, ((0, _NP - _N), (0, 0)))

    zcol = jnp.zeros((_RPT,), jnp.float32)
    onesv = jnp.ones((_B,), jnp.float32)
    zrows = jnp.zeros((_RPT, _D), jnp.float32)

    deg = _deg_kernel(colp, zcol, onesv)
    d0 = deg[:, None]

    x0, hs, dv = _init_call(x_p, W0, b0[None, :], d0)

    b1r = b1[None, :]
    for layer in range(_L):
        s = _spmm_kernel(hs, idxp, zrows)
        beta = float(np.log(_THETA / (layer + 1) + 1.0))
        hs = _layer_call(beta, layer == _L - 1, s, hs, x0, dv,
                         convW[layer], W1, b1r)
    return hs[:_N]


# speedup vs baseline: 1.4935x; 1.3494x over previous
import functools

import jax
import jax.numpy as jnp
from jax import lax
from jax.experimental import pallas as pl
from jax.experimental.pallas import tpu as pltpu
from jax.experimental.pallas import tpu_sc as plsc

_N = 10000
_E = 320000
_D = 128
_L = 4
_ALPHA = 0.1
_THETA = 0.5

_NC = 2
_NS = 16
_NW = _NC * _NS
_B = 128

_NP = 10240
_PT = 10112
_EP = _PT * _NW
_NB = _PT // _B
_RPT = _NP // _NS

_BN = 1024
_GRID = _NP // _BN


def _mesh():
    return plsc.VectorSubcoreMesh(
        core_axis_name="c", subcore_axis_name="s",
        num_cores=_NC, num_subcores=_NS)


@functools.partial(
    pl.kernel,
    out_type=jax.ShapeDtypeStruct((_NC, _NP), jnp.float32),
    mesh=_mesh(),
    scratch_types=[
        pltpu.VMEM_SHARED((_NP,), jnp.float32),
        pltpu.VMEM((_B,), jnp.int32),
        pltpu.VMEM((_B,), jnp.float32),
        pltpu.SemaphoreType.DMA,
    ],
)
def _deg_kernel(colp, zcol, onesv, out, acc, colb, oneb, sem):
    c = lax.axis_index("c")
    s = lax.axis_index("s")
    w = s * _NC + c
    pltpu.sync_copy(zcol, acc.at[pl.ds(s * _RPT, _RPT)])
    pltpu.sync_copy(onesv, oneb)
    plsc.subcore_barrier()
    base = w * _PT

    def body(b, carry):
        off = base + b * _B
        pltpu.sync_copy(colp.at[pl.ds(off, _B)], colb)
        pltpu.sync_copy(oneb, acc.at[colb], add=True)
        return carry

    lax.fori_loop(0, _NB, body, 0)
    plsc.subcore_barrier()
    pltpu.sync_copy(acc.at[pl.ds(s * _RPT, _RPT)],
                    out.at[c, pl.ds(s * _RPT, _RPT)])


@functools.partial(
    pl.kernel,
    out_type=jax.ShapeDtypeStruct((_NC, _NP, _D), jnp.float32),
    mesh=_mesh(),
    scratch_types=[
        pltpu.VMEM_SHARED((_NP, _D), jnp.float32),
        pltpu.VMEM((_B, _D), jnp.float32),
        pltpu.VMEM((_B,), jnp.int32),
        pltpu.VMEM((_B,), jnp.int32),
        pltpu.SemaphoreType.DMA,
    ],
)
def _spmm_kernel(hs, rowp, colp, zrows, out, acc, rows, colb, rowb, sem):
    c = lax.axis_index("c")
    s = lax.axis_index("s")
    w = s * _NC + c
    pltpu.sync_copy(zrows, acc.at[pl.ds(s * _RPT, _RPT)])
    plsc.subcore_barrier()
    base = w * _PT

    def body(b, carry):
        off = base + b * _B
        pltpu.sync_copy(colp.at[pl.ds(off, _B)], colb)
        pltpu.async_copy(hs.at[colb], rows, sem).wait()
        pltpu.sync_copy(rowp.at[pl.ds(off, _B)], rowb)
        pltpu.sync_copy(rows, acc.at[rowb], add=True)
        return carry

    lax.fori_loop(0, _NB, body, 0)
    plsc.subcore_barrier()
    pltpu.sync_copy(acc.at[pl.ds(s * _RPT, _RPT)],
                    out.at[c, pl.ds(s * _RPT, _RPT)])


def _init_body(x_ref, w0_ref, b0_ref, d0_ref, d1_ref, x0_ref, hs_ref, dv_ref):
    h = jnp.dot(x_ref[...], w0_ref[...],
                preferred_element_type=jnp.float32,
                precision=lax.Precision.HIGHEST)
    h = jnp.maximum(h + b0_ref[...], 0.0)
    dv = lax.rsqrt(d0_ref[...] + d1_ref[...] + 1.0)
    x0_ref[...] = h
    hs_ref[...] = h * dv
    dv_ref[...] = dv


def _layer_body(beta, last, s2_ref, hs_ref, x0_ref, dv_ref, w_ref,
                w1_ref, b1_ref, out_ref):
    s = s2_ref[0] + s2_ref[1]
    agg = (s + hs_ref[...]) * dv_ref[...]
    z = (1.0 - _ALPHA) * agg + _ALPHA * x0_ref[...]
    zw = jnp.dot(z, w_ref[...], preferred_element_type=jnp.float32,
                 precision=lax.Precision.HIGHEST)
    h = jnp.maximum((1.0 - beta) * z + beta * zw, 0.0)
    if last:
        y = jnp.dot(h, w1_ref[...], preferred_element_type=jnp.float32,
                    precision=lax.Precision.HIGHEST)
        out_ref[...] = y + b1_ref[...]
    else:
        out_ref[...] = h * dv_ref[...]


_ROWS = pl.BlockSpec((_BN, _D), lambda i: (i, 0))
_COL1 = pl.BlockSpec((_BN, 1), lambda i: (i, 0))
_FULL = pl.BlockSpec((_D, _D), lambda i: (0, 0))
_BIAS = pl.BlockSpec((1, _D), lambda i: (0, 0))
_S2 = pl.BlockSpec((_NC, _BN, _D), lambda i: (0, i, 0))


def _init_call(x_p, W0, b0, d0, d1):
    return pl.pallas_call(
        _init_body,
        grid=(_GRID,),
        in_specs=[_ROWS, _FULL, _BIAS, _COL1, _COL1],
        out_specs=[_ROWS, _ROWS, _COL1],
        out_shape=[
            jax.ShapeDtypeStruct((_NP, _D), jnp.float32),
            jax.ShapeDtypeStruct((_NP, _D), jnp.float32),
            jax.ShapeDtypeStruct((_NP, 1), jnp.float32),
        ],
    )(x_p, W0, b0, d0, d1)


def _layer_call(beta, last, s2, hs, x0, dv, W, W1, b1):
    return pl.pallas_call(
        functools.partial(_layer_body, beta, last),
        grid=(_GRID,),
        in_specs=[_S2, _ROWS, _ROWS, _COL1, _FULL, _FULL, _BIAS],
        out_specs=_ROWS,
        out_shape=jax.ShapeDtypeStruct((_NP, _D), jnp.float32),
    )(s2, hs, x0, dv, W, W1, b1)


def kernel(x, edge_index, W0, b0, convW, W1, b1):
    import numpy as np

    row = edge_index[0]
    col = edge_index[1]
    pad = jnp.full((_EP - _E,), _N, dtype=jnp.int32)
    rowp = jnp.concatenate([row, pad])
    colp = jnp.concatenate([col, pad])
    x_p = jnp.pad(x, ((0, _NP - _N), (0, 0)))

    zcol = jnp.zeros((_RPT,), jnp.float32)
    onesv = jnp.ones((_B,), jnp.float32)
    zrows = jnp.zeros((_RPT, _D), jnp.float32)

    deg = _deg_kernel(colp, zcol, onesv)
    d0 = deg[0][:, None]
    d1 = deg[1][:, None]

    x0, hs, dv = _init_call(x_p, W0, b0[None, :], d0, d1)

    b1r = b1[None, :]
    for layer in range(_L):
        s2 = _spmm_kernel(hs, rowp, colp, zrows)
        beta = float(np.log(_THETA / (layer + 1) + 1.0))
        hs = _layer_call(beta, layer == _L - 1, s2, hs, x0, dv,
                         convW[layer], W1, b1r)
    return hs[:_N]


# speedup vs baseline: 1.6491x; 1.1042x over previous
import functools

import jax
import jax.numpy as jnp
from jax import lax
from jax.experimental import pallas as pl
from jax.experimental.pallas import tpu as pltpu
from jax.experimental.pallas import tpu_sc as plsc

_N = 10000
_E = 320000
_D = 128
_L = 4
_ALPHA = 0.1
_THETA = 0.5

_NC = 2
_NS = 16
_NW = _NC * _NS
_B = 128

_NP = 10240
_PT = 10112
_EP = _PT * _NW
_NB = _PT // _B
_RPT = _NP // _NS

_BN = 1024
_GRID = _NP // _BN


def _mesh():
    return plsc.VectorSubcoreMesh(
        core_axis_name="c", subcore_axis_name="s",
        num_cores=_NC, num_subcores=_NS)


@functools.partial(
    pl.kernel,
    out_type=jax.ShapeDtypeStruct((_NC, _NP), jnp.float32),
    mesh=_mesh(),
    scratch_types=[
        pltpu.VMEM_SHARED((_NP,), jnp.float32),
        pltpu.VMEM((_B,), jnp.int32),
        pltpu.VMEM((_B,), jnp.float32),
        pltpu.SemaphoreType.DMA,
    ],
)
def _deg_kernel(colp, zcol, onesv, out, acc, colb, oneb, sem):
    c = lax.axis_index("c")
    s = lax.axis_index("s")
    w = s * _NC + c
    pltpu.sync_copy(zcol, acc.at[pl.ds(s * _RPT, _RPT)])
    pltpu.sync_copy(onesv, oneb)
    plsc.subcore_barrier()
    base = w * _PT

    def body(b, carry):
        off = base + b * _B
        pltpu.sync_copy(colp.at[pl.ds(off, _B)], colb)
        pltpu.sync_copy(oneb, acc.at[colb], add=True)
        return carry

    lax.fori_loop(0, _NB, body, 0)
    plsc.subcore_barrier()
    pltpu.sync_copy(acc.at[pl.ds(s * _RPT, _RPT)],
                    out.at[c, pl.ds(s * _RPT, _RPT)])


_NB0 = 96
_NB1 = 2 * _NB - _NB0


@functools.partial(
    pl.kernel,
    out_type=jax.ShapeDtypeStruct((_NC, _NP, _D), jnp.float32),
    mesh=_mesh(),
    scratch_types=[
        pltpu.VMEM_SHARED((_NP, _D), jnp.float32),
        pltpu.VMEM((_B, _D), jnp.float32),
        pltpu.VMEM((_B,), jnp.int32),
        pltpu.VMEM((_B,), jnp.int32),
        pltpu.SemaphoreType.DMA,
    ],
)
def _spmm_kernel(hs, rowp, colp, zrows, out, acc, rows, colb, rowb, sem):
    c = lax.axis_index("c")
    s = lax.axis_index("s")
    pltpu.sync_copy(zrows, acc.at[pl.ds(s * _RPT, _RPT)])
    plsc.subcore_barrier()
    stride = (_NB0 + _NB1) * _B

    def make_body(base):
        def body(b, carry):
            off = base + b * _B
            pltpu.sync_copy(colp.at[pl.ds(off, _B)], colb)
            pltpu.async_copy(hs.at[colb], rows, sem).wait()
            pltpu.sync_copy(rowp.at[pl.ds(off, _B)], rowb)
            pltpu.sync_copy(rows, acc.at[rowb], add=True)
            return carry
        return body

    @pl.when(c == 0)
    def _():
        lax.fori_loop(0, _NB0, make_body(s * stride), 0)

    @pl.when(c == 1)
    def _():
        lax.fori_loop(0, _NB1, make_body(s * stride + _NB0 * _B), 0)

    plsc.subcore_barrier()
    pltpu.sync_copy(acc.at[pl.ds(s * _RPT, _RPT)],
                    out.at[c, pl.ds(s * _RPT, _RPT)])


def _init_body(x_ref, w0_ref, b0_ref, d0_ref, d1_ref, x0_ref, hs_ref, dv_ref):
    h = jnp.dot(x_ref[...], w0_ref[...],
                preferred_element_type=jnp.float32,
                precision=lax.Precision.HIGHEST)
    h = jnp.maximum(h + b0_ref[...], 0.0)
    dv = lax.rsqrt(d0_ref[...] + d1_ref[...] + 1.0)
    x0_ref[...] = h
    hs_ref[...] = h * dv
    dv_ref[...] = dv


def _layer_body(beta, last, s2_ref, hs_ref, x0_ref, dv_ref, w_ref,
                w1_ref, b1_ref, out_ref):
    s = s2_ref[0] + s2_ref[1]
    agg = (s + hs_ref[...]) * dv_ref[...]
    z = (1.0 - _ALPHA) * agg + _ALPHA * x0_ref[...]
    zw = jnp.dot(z, w_ref[...], preferred_element_type=jnp.float32,
                 precision=lax.Precision.HIGHEST)
    h = jnp.maximum((1.0 - beta) * z + beta * zw, 0.0)
    if last:
        y = jnp.dot(h, w1_ref[...], preferred_element_type=jnp.float32,
                    precision=lax.Precision.HIGHEST)
        out_ref[...] = y + b1_ref[...]
    else:
        out_ref[...] = h * dv_ref[...]


_ROWS = pl.BlockSpec((_BN, _D), lambda i: (i, 0))
_COL1 = pl.BlockSpec((_BN, 1), lambda i: (i, 0))
_FULL = pl.BlockSpec((_D, _D), lambda i: (0, 0))
_BIAS = pl.BlockSpec((1, _D), lambda i: (0, 0))
_S2 = pl.BlockSpec((_NC, _BN, _D), lambda i: (0, i, 0))


def _init_call(x_p, W0, b0, d0, d1):
    return pl.pallas_call(
        _init_body,
        grid=(_GRID,),
        in_specs=[_ROWS, _FULL, _BIAS, _COL1, _COL1],
        out_specs=[_ROWS, _ROWS, _COL1],
        out_shape=[
            jax.ShapeDtypeStruct((_NP, _D), jnp.float32),
            jax.ShapeDtypeStruct((_NP, _D), jnp.float32),
            jax.ShapeDtypeStruct((_NP, 1), jnp.float32),
        ],
    )(x_p, W0, b0, d0, d1)


def _layer_call(beta, last, s2, hs, x0, dv, W, W1, b1):
    return pl.pallas_call(
        functools.partial(_layer_body, beta, last),
        grid=(_GRID,),
        in_specs=[_S2, _ROWS, _ROWS, _COL1, _FULL, _FULL, _BIAS],
        out_specs=_ROWS,
        out_shape=jax.ShapeDtypeStruct((_NP, _D), jnp.float32),
    )(s2, hs, x0, dv, W, W1, b1)


def kernel(x, edge_index, W0, b0, convW, W1, b1):
    import numpy as np

    row = edge_index[0]
    col = edge_index[1]
    pad = jnp.full((_EP - _E,), _N, dtype=jnp.int32)
    rowp = jnp.concatenate([row, pad])
    colp = jnp.concatenate([col, pad])
    x_p = jnp.pad(x, ((0, _NP - _N), (0, 0)))

    zcol = jnp.zeros((_RPT,), jnp.float32)
    onesv = jnp.ones((_B,), jnp.float32)
    zrows = jnp.zeros((_RPT, _D), jnp.float32)

    deg = _deg_kernel(colp, zcol, onesv)
    d0 = deg[0][:, None]
    d1 = deg[1][:, None]

    x0, hs, dv = _init_call(x_p, W0, b0[None, :], d0, d1)

    b1r = b1[None, :]
    for layer in range(_L):
        s2 = _spmm_kernel(hs, rowp, colp, zrows)
        beta = float(np.log(_THETA / (layer + 1) + 1.0))
        hs = _layer_call(beta, layer == _L - 1, s2, hs, x0, dv,
                         convW[layer], W1, b1r)
    return hs[:_N]


# speedup vs baseline: 1.6798x; 1.0187x over previous
import functools

import jax
import jax.numpy as jnp
from jax import lax
from jax.experimental import pallas as pl
from jax.experimental.pallas import tpu as pltpu
from jax.experimental.pallas import tpu_sc as plsc

_N = 10000
_E = 320000
_D = 128
_L = 4
_ALPHA = 0.1
_THETA = 0.5

_NC = 2
_NS = 16
_NW = _NC * _NS
_B = 128

_NP = 10240
_PT = 10112
_EP = _PT * _NW
_NB = _PT // _B
_RPT = _NP // _NS

_BN = 1024
_GRID = _NP // _BN


def _mesh():
    return plsc.VectorSubcoreMesh(
        core_axis_name="c", subcore_axis_name="s",
        num_cores=_NC, num_subcores=_NS)


@functools.partial(
    pl.kernel,
    out_type=jax.ShapeDtypeStruct((_NC, _NP), jnp.float32),
    mesh=_mesh(),
    scratch_types=[
        pltpu.VMEM_SHARED((_NP,), jnp.float32),
        pltpu.VMEM((_B,), jnp.int32),
        pltpu.VMEM((_B,), jnp.float32),
        pltpu.SemaphoreType.DMA,
    ],
)
def _deg_kernel(colp, zcol, onesv, out, acc, colb, oneb, sem):
    c = lax.axis_index("c")
    s = lax.axis_index("s")
    w = s * _NC + c
    pltpu.sync_copy(zcol, acc.at[pl.ds(s * _RPT, _RPT)])
    pltpu.sync_copy(onesv, oneb)
    plsc.subcore_barrier()
    base = w * _PT

    def body(b, carry):
        off = base + b * _B
        pltpu.sync_copy(colp.at[pl.ds(off, _B)], colb)
        pltpu.sync_copy(oneb, acc.at[colb], add=True)
        return carry

    lax.fori_loop(0, _NB, body, 0)
    plsc.subcore_barrier()
    pltpu.sync_copy(acc.at[pl.ds(s * _RPT, _RPT)],
                    out.at[c, pl.ds(s * _RPT, _RPT)])


_NB0 = 102
_NB1 = 2 * _NB - _NB0


@functools.partial(
    pl.kernel,
    out_type=jax.ShapeDtypeStruct((_NC, _NP, _D), jnp.float32),
    mesh=_mesh(),
    scratch_types=[
        pltpu.VMEM_SHARED((_NP, _D), jnp.float32),
        pltpu.VMEM((_B, _D), jnp.float32),
        pltpu.VMEM((_B,), jnp.int32),
        pltpu.VMEM((_B,), jnp.int32),
        pltpu.SemaphoreType.DMA,
    ],
)
def _spmm_kernel(hs, rowp, colp, zrows, out, acc, rows, colb, rowb, sem):
    c = lax.axis_index("c")
    s = lax.axis_index("s")
    pltpu.sync_copy(zrows, acc.at[pl.ds(s * _RPT, _RPT)])
    plsc.subcore_barrier()
    stride = (_NB0 + _NB1) * _B

    def make_body(base):
        def body(b, carry):
            off = base + b * _B
            pltpu.sync_copy(colp.at[pl.ds(off, _B)], colb)
            pltpu.async_copy(hs.at[colb], rows, sem).wait()
            pltpu.sync_copy(rowp.at[pl.ds(off, _B)], rowb)
            pltpu.sync_copy(rows, acc.at[rowb], add=True)
            return carry
        return body

    @pl.when(c == 0)
    def _():
        lax.fori_loop(0, _NB0, make_body(s * stride), 0)

    @pl.when(c == 1)
    def _():
        lax.fori_loop(0, _NB1, make_body(s * stride + _NB0 * _B), 0)

    plsc.subcore_barrier()
    pltpu.sync_copy(acc.at[pl.ds(s * _RPT, _RPT)],
                    out.at[c, pl.ds(s * _RPT, _RPT)])


def _init_body(x_ref, w0_ref, b0_ref, d0_ref, d1_ref, x0_ref, hs_ref, dv_ref):
    h = jnp.dot(x_ref[...], w0_ref[...],
                preferred_element_type=jnp.float32,
                precision=lax.Precision.HIGHEST)
    h = jnp.maximum(h + b0_ref[...], 0.0)
    dv = lax.rsqrt(d0_ref[...] + d1_ref[...] + 1.0)
    x0_ref[...] = h
    hs_ref[...] = h * dv
    dv_ref[...] = dv


def _layer_body(beta, last, s2_ref, hs_ref, x0_ref, dv_ref, w_ref,
                w1_ref, b1_ref, out_ref):
    s = s2_ref[0] + s2_ref[1]
    agg = (s + hs_ref[...]) * dv_ref[...]
    z = (1.0 - _ALPHA) * agg + _ALPHA * x0_ref[...]
    zw = jnp.dot(z, w_ref[...], preferred_element_type=jnp.float32,
                 precision=lax.Precision.HIGHEST)
    h = jnp.maximum((1.0 - beta) * z + beta * zw, 0.0)
    if last:
        y = jnp.dot(h, w1_ref[...], preferred_element_type=jnp.float32,
                    precision=lax.Precision.HIGHEST)
        out_ref[...] = y + b1_ref[...]
    else:
        out_ref[...] = h * dv_ref[...]


_ROWS = pl.BlockSpec((_BN, _D), lambda i: (i, 0))
_COL1 = pl.BlockSpec((_BN, 1), lambda i: (i, 0))
_FULL = pl.BlockSpec((_D, _D), lambda i: (0, 0))
_BIAS = pl.BlockSpec((1, _D), lambda i: (0, 0))
_S2 = pl.BlockSpec((_NC, _BN, _D), lambda i: (0, i, 0))


def _init_call(x_p, W0, b0, d0, d1):
    return pl.pallas_call(
        _init_body,
        grid=(_GRID,),
        in_specs=[_ROWS, _FULL, _BIAS, _COL1, _COL1],
        out_specs=[_ROWS, _ROWS, _COL1],
        out_shape=[
            jax.ShapeDtypeStruct((_NP, _D), jnp.float32),
            jax.ShapeDtypeStruct((_NP, _D), jnp.float32),
            jax.ShapeDtypeStruct((_NP, 1), jnp.float32),
        ],
    )(x_p, W0, b0, d0, d1)


def _layer_call(beta, last, s2, hs, x0, dv, W, W1, b1):
    return pl.pallas_call(
        functools.partial(_layer_body, beta, last),
        grid=(_GRID,),
        in_specs=[_S2, _ROWS, _ROWS, _COL1, _FULL, _FULL, _BIAS],
        out_specs=_ROWS,
        out_shape=jax.ShapeDtypeStruct((_NP, _D), jnp.float32),
    )(s2, hs, x0, dv, W, W1, b1)


def kernel(x, edge_index, W0, b0, convW, W1, b1):
    import numpy as np

    row = edge_index[0]
    col = edge_index[1]
    pad = jnp.full((_EP - _E,), _N, dtype=jnp.int32)
    rowp = jnp.concatenate([row, pad])
    colp = jnp.concatenate([col, pad])
    x_p = jnp.pad(x, ((0, _NP - _N), (0, 0)))

    zcol = jnp.zeros((_RPT,), jnp.float32)
    onesv = jnp.ones((_B,), jnp.float32)
    zrows = jnp.zeros((_RPT, _D), jnp.float32)

    deg = _deg_kernel(colp, zcol, onesv)
    d0 = deg[0][:, None]
    d1 = deg[1][:, None]

    x0, hs, dv = _init_call(x_p, W0, b0[None, :], d0, d1)

    b1r = b1[None, :]
    for layer in range(_L):
        s2 = _spmm_kernel(hs, rowp, colp, zrows)
        beta = float(np.log(_THETA / (layer + 1) + 1.0))
        hs = _layer_call(beta, layer == _L - 1, s2, hs, x0, dv,
                         convW[layer], W1, b1r)
    return hs[:_N]


# speedup vs baseline: 1.7047x; 1.0148x over previous
import functools

import jax
import jax.numpy as jnp
from jax import lax
from jax.experimental import pallas as pl
from jax.experimental.pallas import tpu as pltpu
from jax.experimental.pallas import tpu_sc as plsc

_N = 10000
_E = 320000
_D = 128
_L = 4
_ALPHA = 0.1
_THETA = 0.5

_NC = 2
_NS = 16
_NW = _NC * _NS
_B = 128

_NP = 10240
_PT = 10112
_EP = _PT * _NW
_NB = _PT // _B
_RPT = _NP // _NS

_BN = 1024
_GRID = _NP // _BN


def _mesh():
    return plsc.VectorSubcoreMesh(
        core_axis_name="c", subcore_axis_name="s",
        num_cores=_NC, num_subcores=_NS)


_NBD = _EP // _B // _NS


@functools.partial(
    pl.kernel,
    out_type=jax.ShapeDtypeStruct((_NP,), jnp.float32),
    mesh=_mesh(),
    scratch_types=[
        pltpu.VMEM_SHARED((_NP,), jnp.float32),
        pltpu.VMEM((_NBD, _B), jnp.int32),
        pltpu.VMEM((_B,), jnp.float32),
        pltpu.SemaphoreType.DMA,
    ],
)
def _deg_kernel(colp, zcol, onesv, out, acc, col_all, oneb, sem):
    c = lax.axis_index("c")
    s = lax.axis_index("s")

    @pl.when(c == 0)
    def _():
        pltpu.sync_copy(zcol, acc.at[pl.ds(s * _RPT, _RPT)])
        pltpu.sync_copy(onesv, oneb)
        pltpu.sync_copy(colp.at[s], col_all)
        plsc.subcore_barrier()
        descs = [pltpu.async_copy(oneb, acc.at[col_all.at[b]], sem, add=True)
                 for b in range(_NBD)]
        for d in descs:
            d.wait()
        plsc.subcore_barrier()
        pltpu.sync_copy(acc.at[pl.ds(s * _RPT, _RPT)],
                        out.at[pl.ds(s * _RPT, _RPT)])


_NB0 = 102
_NB1 = 2 * _NB - _NB0


@functools.partial(
    pl.kernel,
    out_type=jax.ShapeDtypeStruct((_NC, _NP, _D), jnp.float32),
    mesh=_mesh(),
    scratch_types=[
        pltpu.VMEM_SHARED((_NP, _D), jnp.float32),
        pltpu.VMEM((_B, _D), jnp.float32),
        pltpu.VMEM((_B,), jnp.int32),
        pltpu.VMEM((_B,), jnp.int32),
        pltpu.SemaphoreType.DMA,
    ],
)
def _spmm_kernel(hs, rowp, colp, zrows, out, acc, rows, colb, rowb, sem):
    c = lax.axis_index("c")
    s = lax.axis_index("s")
    pltpu.sync_copy(zrows, acc.at[pl.ds(s * _RPT, _RPT)])
    plsc.subcore_barrier()
    stride = (_NB0 + _NB1) * _B

    def make_body(base):
        def body(b, carry):
            off = base + b * _B
            pltpu.sync_copy(colp.at[pl.ds(off, _B)], colb)
            pltpu.async_copy(hs.at[colb], rows, sem).wait()
            pltpu.sync_copy(rowp.at[pl.ds(off, _B)], rowb)
            pltpu.sync_copy(rows, acc.at[rowb], add=True)
            return carry
        return body

    @pl.when(c == 0)
    def _():
        lax.fori_loop(0, _NB0, make_body(s * stride), 0)

    @pl.when(c == 1)
    def _():
        lax.fori_loop(0, _NB1, make_body(s * stride + _NB0 * _B), 0)

    plsc.subcore_barrier()
    pltpu.sync_copy(acc.at[pl.ds(s * _RPT, _RPT)],
                    out.at[c, pl.ds(s * _RPT, _RPT)])


def _init_body(x_ref, w0_ref, b0_ref, d0_ref, x0_ref, hs_ref, dv_ref):
    h = jnp.dot(x_ref[...], w0_ref[...],
                preferred_element_type=jnp.float32,
                precision=lax.Precision.HIGHEST)
    h = jnp.maximum(h + b0_ref[...], 0.0)
    dv = lax.rsqrt(d0_ref[...] + 1.0)
    x0_ref[...] = h
    hs_ref[...] = h * dv
    dv_ref[...] = dv


def _layer_body(beta, last, s2_ref, hs_ref, x0_ref, dv_ref, w_ref,
                w1_ref, b1_ref, out_ref):
    s = s2_ref[0] + s2_ref[1]
    agg = (s + hs_ref[...]) * dv_ref[...]
    z = (1.0 - _ALPHA) * agg + _ALPHA * x0_ref[...]
    zw = jnp.dot(z, w_ref[...], preferred_element_type=jnp.float32,
                 precision=lax.Precision.HIGHEST)
    h = jnp.maximum((1.0 - beta) * z + beta * zw, 0.0)
    if last:
        y = jnp.dot(h, w1_ref[...], preferred_element_type=jnp.float32,
                    precision=lax.Precision.HIGHEST)
        out_ref[...] = y + b1_ref[...]
    else:
        out_ref[...] = h * dv_ref[...]


_ROWS = pl.BlockSpec((_BN, _D), lambda i: (i, 0))
_COL1 = pl.BlockSpec((_BN, 1), lambda i: (i, 0))
_FULL = pl.BlockSpec((_D, _D), lambda i: (0, 0))
_BIAS = pl.BlockSpec((1, _D), lambda i: (0, 0))
_S2 = pl.BlockSpec((_NC, _BN, _D), lambda i: (0, i, 0))


def _init_call(x_p, W0, b0, d0):
    return pl.pallas_call(
        _init_body,
        grid=(_GRID,),
        in_specs=[_ROWS, _FULL, _BIAS, _COL1],
        out_specs=[_ROWS, _ROWS, _COL1],
        out_shape=[
            jax.ShapeDtypeStruct((_NP, _D), jnp.float32),
            jax.ShapeDtypeStruct((_NP, _D), jnp.float32),
            jax.ShapeDtypeStruct((_NP, 1), jnp.float32),
        ],
    )(x_p, W0, b0, d0)


def _layer_call(beta, last, s2, hs, x0, dv, W, W1, b1):
    return pl.pallas_call(
        functools.partial(_layer_body, beta, last),
        grid=(_GRID,),
        in_specs=[_S2, _ROWS, _ROWS, _COL1, _FULL, _FULL, _BIAS],
        out_specs=_ROWS,
        out_shape=jax.ShapeDtypeStruct((_NP, _D), jnp.float32),
    )(s2, hs, x0, dv, W, W1, b1)


def kernel(x, edge_index, W0, b0, convW, W1, b1):
    import numpy as np

    row = edge_index[0]
    col = edge_index[1]
    pad = jnp.full((_EP - _E,), _N, dtype=jnp.int32)
    rowp = jnp.concatenate([row, pad])
    colp = jnp.concatenate([col, pad])
    cold = colp.reshape(_NS, _NBD, _B)
    x_p = jnp.pad(x, ((0, _NP - _N), (0, 0)))

    zcol = jnp.zeros((_RPT,), jnp.float32)
    onesv = jnp.ones((_B,), jnp.float32)
    zrows = jnp.zeros((_RPT, _D), jnp.float32)

    deg = _deg_kernel(cold, zcol, onesv)
    d0 = deg[:, None]

    x0, hs, dv = _init_call(x_p, W0, b0[None, :], d0)

    b1r = b1[None, :]
    for layer in range(_L):
        s2 = _spmm_kernel(hs, rowp, colp, zrows)
        beta = float(np.log(_THETA / (layer + 1) + 1.0))
        hs = _layer_call(beta, layer == _L - 1, s2, hs, x0, dv,
                         convW[layer], W1, b1r)
    return hs[:_N]
